# baseline (speedup 1.0000x reference)
import functools

import jax
import jax.numpy as jnp
from jax import lax
from jax.experimental import pallas as pl
from jax.experimental.pallas import tpu as pltpu
from jax.experimental.pallas import tpu_sc as plsc

N_TOTAL = 16777216
N_BINS = 15
NC, NS, L = 2, 16, 16
NW = NC * NS
PER_W = N_TOTAL // NW
CH = 16384
NCH = PER_W // CH
VPC = CH // L


def _sc_body(conf_hbm, pred_hbm, lab_hbm, out_hbm,
             conf0, pred0, lab0, conf1, pred1, lab1, tab, red, sem0, sem1):
    wid = lax.axis_index("s") * NC + lax.axis_index("c")
    base = wid * PER_W

    zero = jnp.zeros((L,), jnp.float32)
    for r in range(3 * L):
        tab[pl.ds(r * L, L)] = zero

    lane = lax.iota(jnp.int32, L)
    ones = jnp.full((L,), 1.0, jnp.float32)

    bufs = ((conf0, pred0, lab0, sem0), (conf1, pred1, lab1, sem1))

    def start(c, b):
        off = base + c * CH
        cb, pb, lb, sem = bufs[b]
        pltpu.async_copy(conf_hbm.at[pl.ds(off, CH)], cb, sem)
        pltpu.async_copy(pred_hbm.at[pl.ds(off, CH)], pb, sem)
        pltpu.async_copy(lab_hbm.at[pl.ds(off, CH)], lb, sem)

    def wait(b):
        cb, pb, lb, sem = bufs[b]
        pltpu.make_async_copy(conf_hbm.at[pl.ds(0, CH)], cb, sem).wait()
        pltpu.make_async_copy(pred_hbm.at[pl.ds(0, CH)], pb, sem).wait()
        pltpu.make_async_copy(lab_hbm.at[pl.ds(0, CH)], lb, sem).wait()

    def process(b):
        cb, pb, lb, _ = bufs[b]

        @plsc.parallel_loop(0, CH, L, unroll=8)
        def body(s):
            c = cb[pl.ds(s, L)]
            p = pb[pl.ds(s, L)]
            lbl = lb[pl.ds(s, L)]
            bi = jnp.minimum((c * 15.0).astype(jnp.int32), 14)
            addr = bi * L + lane
            plsc.addupdate_scatter(tab, [addr], ones)
            plsc.addupdate_scatter(tab, [addr + (L * L)], c)
            plsc.addupdate_scatter(tab, [addr + (2 * L * L)], ones,
                                   mask=p == lbl)

    start(0, 0)

    def outer(k, carry):
        c0 = k * 2
        start(c0 + 1, 1)
        wait(0)
        process(0)

        @pl.when(c0 + 2 < NCH)
        def _():
            start(c0 + 2, 0)

        wait(1)
        process(1)
        return carry

    lax.fori_loop(0, NCH // 2, outer, 0)

    for q in range(3):
        acc_v = jnp.zeros((L,), jnp.float32)
        base_q = q * L * L
        for k in range(L):
            acc_v = acc_v + plsc.load_gather(tab, [lane * L + (base_q + k)])
        red[q] = acc_v
    pltpu.sync_copy(red, out_hbm.at[wid])


_sc_hist = functools.partial(
    pl.kernel,
    mesh=plsc.VectorSubcoreMesh(
        core_axis_name="c", subcore_axis_name="s",
        num_cores=NC, num_subcores=NS),
    out_type=jax.ShapeDtypeStruct((NW, 3, L), jnp.float32),
    compiler_params=pltpu.CompilerParams(needs_layout_passes=False),
    scratch_types=[
        pltpu.VMEM((CH,), jnp.float32),
        pltpu.VMEM((CH,), jnp.int32),
        pltpu.VMEM((CH,), jnp.int32),
        pltpu.VMEM((CH,), jnp.float32),
        pltpu.VMEM((CH,), jnp.int32),
        pltpu.VMEM((CH,), jnp.int32),
        pltpu.VMEM((3 * L * L,), jnp.float32),
        pltpu.VMEM((3, L), jnp.float32),
        pltpu.SemaphoreType.DMA,
        pltpu.SemaphoreType.DMA,
    ],
)(_sc_body)


def _fin_body(p_ref, o_ref):
    cnt = jnp.sum(p_ref[0], axis=0, keepdims=True)
    sconf = jnp.sum(p_ref[1], axis=0, keepdims=True)
    sacc = jnp.sum(p_ref[2], axis=0, keepdims=True)
    denom = jnp.maximum(cnt, 1.0)
    contrib = jnp.abs(sconf / denom - sacc / denom) * (cnt / N_TOTAL)
    contrib = jnp.where(cnt > 0, contrib, 0.0)
    o_ref[0] = jnp.sum(contrib)


_finalize = pl.pallas_call(
    _fin_body,
    out_shape=jax.ShapeDtypeStruct((1,), jnp.float32),
    out_specs=pl.BlockSpec(memory_space=pltpu.SMEM),
)


def kernel(confidences, predictions, labels, title):
    partials = _sc_hist(confidences, predictions, labels)
    return _finalize(jnp.transpose(partials, (1, 0, 2)))


# speedup vs baseline: 1.0222x; 1.0222x over previous
import functools

import jax
import jax.numpy as jnp
from jax import lax
from jax.experimental import pallas as pl
from jax.experimental.pallas import tpu as pltpu
from jax.experimental.pallas import tpu_sc as plsc

N_TOTAL = 16777216
N_BINS = 15
NC, NS, L = 2, 16, 16
NW = NC * NS
PER_W = N_TOTAL // NW
CH = 8192
NCH = PER_W // CH
NBUF = 3


def _sc_body(conf_hbm, pred_hbm, lab_hbm, out_hbm,
             conf0, pred0, lab0, conf1, pred1, lab1, conf2, pred2, lab2,
             tabc, tabf, taba, red, sem0, sem1, sem2):
    wid = lax.axis_index("s") * NC + lax.axis_index("c")
    base = wid * PER_W

    zero = jnp.zeros((L,), jnp.float32)
    for t in (tabc, tabf, taba):
        for r in range(L):
            t[pl.ds(r * L, L)] = zero

    lane = lax.iota(jnp.int32, L)
    ones = jnp.full((L,), 1.0, jnp.float32)

    bufs = ((conf0, pred0, lab0, sem0), (conf1, pred1, lab1, sem1),
            (conf2, pred2, lab2, sem2))

    def start(c, b):
        off = base + c * CH
        cb, pb, lb, sem = bufs[b]
        pltpu.async_copy(conf_hbm.at[pl.ds(off, CH)], cb, sem)
        pltpu.async_copy(pred_hbm.at[pl.ds(off, CH)], pb, sem)
        pltpu.async_copy(lab_hbm.at[pl.ds(off, CH)], lb, sem)

    def wait(b):
        cb, pb, lb, sem = bufs[b]
        pltpu.make_async_copy(conf_hbm.at[pl.ds(0, CH)], cb, sem).wait()
        pltpu.make_async_copy(pred_hbm.at[pl.ds(0, CH)], pb, sem).wait()
        pltpu.make_async_copy(lab_hbm.at[pl.ds(0, CH)], lb, sem).wait()

    def process(b):
        cb, pb, lb, _ = bufs[b]

        @plsc.parallel_loop(0, CH, L, unroll=8)
        def body(s):
            c = cb[pl.ds(s, L)]
            p = pb[pl.ds(s, L)]
            lbl = lb[pl.ds(s, L)]
            addr = (c * 15.0).astype(jnp.int32) * L + lane
            plsc.addupdate_scatter(tabc, [addr], ones)
            plsc.addupdate_scatter(tabf, [addr], c)
            plsc.addupdate_scatter(taba, [addr], ones, mask=p == lbl)

    for c in range(NBUF - 1):
        start(c, c)

    def outer(k, carry):
        c0 = k * NBUF
        for j in range(NBUF):
            c = c0 + j
            b = j
            wait(b)

            @pl.when(c + NBUF - 1 < NCH)
            def _():
                start(c + NBUF - 1, (b + NBUF - 1) % NBUF)

            process(b)
        return carry

    lax.fori_loop(0, NCH // NBUF, outer, 0)
    for c in range((NCH // NBUF) * NBUF, NCH):
        b = c % NBUF
        wait(b)
        process(b)

    for q, t in enumerate((tabc, tabf, taba)):
        acc_v = jnp.zeros((L,), jnp.float32)
        for k in range(L):
            acc_v = acc_v + plsc.load_gather(t, [lane * L + k])
        red[q] = acc_v
    pltpu.sync_copy(red, out_hbm.at[wid])


_sc_hist = functools.partial(
    pl.kernel,
    mesh=plsc.VectorSubcoreMesh(
        core_axis_name="c", subcore_axis_name="s",
        num_cores=NC, num_subcores=NS),
    out_type=jax.ShapeDtypeStruct((NW, 3, L), jnp.float32),
    compiler_params=pltpu.CompilerParams(needs_layout_passes=False),
    scratch_types=(
        [pltpu.VMEM((CH,), jnp.float32),
         pltpu.VMEM((CH,), jnp.int32),
         pltpu.VMEM((CH,), jnp.int32)] * NBUF
        + [pltpu.VMEM((L * L,), jnp.float32)] * 3
        + [pltpu.VMEM((3, L), jnp.float32)]
        + [pltpu.SemaphoreType.DMA] * NBUF
    ),
)(_sc_body)


def _fin_body(p_ref, o_ref):
    cnt = jnp.sum(p_ref[0], axis=0, keepdims=True)
    sconf = jnp.sum(p_ref[1], axis=0, keepdims=True)
    sacc = jnp.sum(p_ref[2], axis=0, keepdims=True)
    denom = jnp.maximum(cnt, 1.0)
    contrib = jnp.abs(sconf / denom - sacc / denom) * (cnt / N_TOTAL)
    contrib = jnp.where(cnt > 0, contrib, 0.0)
    o_ref[0] = jnp.sum(contrib)


_finalize = pl.pallas_call(
    _fin_body,
    out_shape=jax.ShapeDtypeStruct((1,), jnp.float32),
    out_specs=pl.BlockSpec(memory_space=pltpu.SMEM),
)


def kernel(confidences, predictions, labels, title):
    partials = _sc_hist(confidences, predictions, labels)
    return _finalize(jnp.transpose(partials, (1, 0, 2)))
